# baseline (speedup 1.0000x reference)
import functools

import jax
import jax.numpy as jnp
from jax import lax
from jax.experimental import pallas as pl
from jax.experimental.pallas import tpu as pltpu
from jax.experimental.pallas import tpu_sc as plsc

BATCH = 4096
SEQ = 50
EMBED = 128
OUT = 512

NUM_CORES = 2
NUM_SUBCORES = 16
NUM_WORKERS = NUM_CORES * NUM_SUBCORES
BPW = BATCH // NUM_WORKERS
CB = 4
NBUF = 4
NSTEPS = BPW // CB
LANES = 16
NJ = EMBED // LANES
IW = 100
IROWS_PER_TILE = BPW * SEQ // IW

_MESH = plsc.VectorSubcoreMesh(core_axis_name="c", subcore_axis_name="s")


@jax.jit
def _sc_pool(table, idx_flat):
    @functools.partial(
        pl.kernel,
        mesh=_MESH,
        out_type=jax.ShapeDtypeStruct((BATCH, EMBED), jnp.float32),
        scratch_types=[
            pltpu.VMEM((BPW * SEQ,), jnp.int32),
            pltpu.VMEM((NBUF, CB * SEQ, EMBED), jnp.float32),
            pltpu.VMEM((BPW, EMBED), jnp.float32),
        ] + [pltpu.SemaphoreType.DMA] * NBUF,
    )
    def k(table_hbm, idx_hbm, out_hbm, idx_v, rows_v, pooled_v, *sems):
        wid = lax.axis_index("s") * NUM_CORES + lax.axis_index("c")
        pltpu.sync_copy(idx_hbm.at[pl.ds(wid * BPW * SEQ, BPW * SEQ)], idx_v)
        halves = ((0, 104), (104, 96))

        def gather(step, p):
            for off, n in halves:
                pltpu.make_async_copy(
                    table_hbm.at[idx_v.at[pl.ds(step * CB * SEQ + off, n)]],
                    rows_v.at[p, pl.ds(off, n)], sems[p]).start()

        def gather_wait(p):
            for off, n in halves:
                pltpu.make_async_copy(
                    table_hbm.at[idx_v.at[pl.ds(off, n)]],
                    rows_v.at[p, pl.ds(off, n)], sems[p]).wait()

        for p in range(NBUF):
            gather(p, p)

        @pl.loop(0, NSTEPS // NBUF)
        def _step(g):
            for p in range(NBUF):
                step = g * NBUF + p
                gather_wait(p)
                for b in range(CB):
                    def body(t, acc):
                        for u in range(4):
                            l = 1 + t * 4 + u
                            acc = tuple(
                                acc[j] + rows_v[p, b * SEQ + l,
                                                pl.ds(j * LANES, LANES)]
                                for j in range(NJ))
                        return acc
                    acc0 = tuple(
                        rows_v[p, b * SEQ, pl.ds(j * LANES, LANES)]
                        + rows_v[p, b * SEQ + SEQ - 1, pl.ds(j * LANES, LANES)]
                        for j in range(NJ))
                    acc = lax.fori_loop(0, (SEQ - 2) // 4, body, acc0)
                    for j in range(NJ):
                        pooled_v[step * CB + b, pl.ds(j * LANES, LANES)] = acc[j]

                @pl.when(step + NBUF < NSTEPS)
                def _():
                    gather(step + NBUF, p)

        pltpu.sync_copy(pooled_v, out_hbm.at[pl.ds(wid * BPW, BPW)])

    return k(table, idx_flat)


def _tc_proj_body(p_ref, w_ref, b_ref, o_ref):
    acc = jax.lax.dot_general(
        p_ref[...], w_ref[...],
        dimension_numbers=(((1,), (0,)), ((), ())),
        preferred_element_type=jnp.float32)
    o_ref[...] = acc * (1.0 / SEQ) + b_ref[...]


@jax.jit
def _tc_proj(pooled, W, b2d):
    blk = 512
    return pl.pallas_call(
        _tc_proj_body,
        grid=(BATCH // blk,),
        in_specs=[
            pl.BlockSpec((blk, EMBED), lambda i: (i, 0)),
            pl.BlockSpec((EMBED, OUT), lambda i: (0, 0)),
            pl.BlockSpec((1, OUT), lambda i: (0, 0)),
        ],
        out_specs=pl.BlockSpec((blk, OUT), lambda i: (i, 0)),
        out_shape=jax.ShapeDtypeStruct((BATCH, OUT), jnp.float32),
    )(pooled, W, b2d)


def kernel(x, table, W, b):
    idx_flat = x.reshape(-1).astype(jnp.int32)
    pooled = _sc_pool(table, idx_flat)
    return _tc_proj(pooled, W, b.reshape(1, OUT))


# speedup vs baseline: 1.0471x; 1.0471x over previous
import functools

import jax
import jax.numpy as jnp
from jax import lax
from jax.experimental import pallas as pl
from jax.experimental.pallas import tpu as pltpu
from jax.experimental.pallas import tpu_sc as plsc

BATCH = 4096
SEQ = 50
EMBED = 128
OUT = 512

NUM_CORES = 2
NUM_SUBCORES = 16
NUM_WORKERS = NUM_CORES * NUM_SUBCORES
BPW = BATCH // NUM_WORKERS
CB = 4
NBUF = 4
NSTEPS = BPW // CB
LANES = 16
NJ = EMBED // LANES
IW = 100
IROWS_PER_TILE = BPW * SEQ // IW

_MESH = plsc.VectorSubcoreMesh(core_axis_name="c", subcore_axis_name="s")


@jax.jit
def _sc_pool(table, idx_flat):
    @functools.partial(
        pl.kernel,
        mesh=_MESH,
        out_type=jax.ShapeDtypeStruct((BATCH, EMBED), jnp.float32),
        scratch_types=[
            pltpu.VMEM((BPW * SEQ,), jnp.int32),
            pltpu.VMEM((NBUF, CB * SEQ, EMBED), jnp.float32),
            pltpu.VMEM((BPW, EMBED), jnp.float32),
        ] + [pltpu.SemaphoreType.DMA] * NBUF,
    )
    def k(table_hbm, idx_hbm, out_hbm, idx_v, rows_v, pooled_v, *sems):
        wid = lax.axis_index("s") * NUM_CORES + lax.axis_index("c")
        pltpu.sync_copy(idx_hbm.at[pl.ds(wid * BPW * SEQ, BPW * SEQ)], idx_v)
        halves = ((0, 104), (104, 96))

        def gather(step, p):
            for off, n in halves:
                pltpu.make_async_copy(
                    table_hbm.at[idx_v.at[pl.ds(step * CB * SEQ + off, n)]],
                    rows_v.at[p, pl.ds(off, n)], sems[p]).start()

        def gather_wait(p):
            for off, n in halves:
                pltpu.make_async_copy(
                    table_hbm.at[idx_v.at[pl.ds(off, n)]],
                    rows_v.at[p, pl.ds(off, n)], sems[p]).wait()

        for p in range(NBUF):
            gather(p, p)

        @pl.loop(0, NSTEPS // NBUF)
        def _step(g):
            for p in range(NBUF):
                step = g * NBUF + p
                gather_wait(p)

                @pl.loop(0, CB)
                def _elem(b):
                    def body(t, acc):
                        for u in range(4):
                            l = 1 + t * 4 + u
                            acc = tuple(
                                acc[j] + rows_v[p, b * SEQ + l,
                                                pl.ds(j * LANES, LANES)]
                                for j in range(NJ))
                        return acc
                    acc0 = tuple(
                        rows_v[p, b * SEQ, pl.ds(j * LANES, LANES)]
                        + rows_v[p, b * SEQ + SEQ - 1, pl.ds(j * LANES, LANES)]
                        for j in range(NJ))
                    acc = lax.fori_loop(0, (SEQ - 2) // 4, body, acc0)
                    for j in range(NJ):
                        pooled_v[step * CB + b, pl.ds(j * LANES, LANES)] = acc[j]

                @pl.when(step + NBUF < NSTEPS)
                def _():
                    gather(step + NBUF, p)

        pltpu.sync_copy(pooled_v, out_hbm.at[pl.ds(wid * BPW, BPW)])

    return k(table, idx_flat)


def _tc_proj_body(p_ref, w_ref, b_ref, o_ref):
    acc = jax.lax.dot_general(
        p_ref[...], w_ref[...],
        dimension_numbers=(((1,), (0,)), ((), ())),
        preferred_element_type=jnp.float32)
    o_ref[...] = acc * (1.0 / SEQ) + b_ref[...]


@jax.jit
def _tc_proj(pooled, W, b2d):
    blk = 1024
    return pl.pallas_call(
        _tc_proj_body,
        grid=(BATCH // blk,),
        in_specs=[
            pl.BlockSpec((blk, EMBED), lambda i: (i, 0)),
            pl.BlockSpec((EMBED, OUT), lambda i: (0, 0)),
            pl.BlockSpec((1, OUT), lambda i: (0, 0)),
        ],
        out_specs=pl.BlockSpec((blk, OUT), lambda i: (i, 0)),
        out_shape=jax.ShapeDtypeStruct((BATCH, OUT), jnp.float32),
    )(pooled, W, b2d)


def kernel(x, table, W, b):
    idx_flat = x.reshape(-1).astype(jnp.int32)
    pooled = _sc_pool(table, idx_flat)
    return _tc_proj(pooled, W, b.reshape(1, OUT))


# speedup vs baseline: 1.0598x; 1.0121x over previous
import functools

import jax
import jax.numpy as jnp
from jax import lax
from jax.experimental import pallas as pl
from jax.experimental.pallas import tpu as pltpu
from jax.experimental.pallas import tpu_sc as plsc

BATCH = 4096
SEQ = 50
EMBED = 128
OUT = 512

NUM_CORES = 2
NUM_SUBCORES = 16
NUM_WORKERS = NUM_CORES * NUM_SUBCORES
BPW = BATCH // NUM_WORKERS
CB = 4
NBUF = 4
NSTEPS = BPW // CB
LANES = 16
NJ = EMBED // LANES
IW = 100
IROWS_PER_TILE = BPW * SEQ // IW

_MESH = plsc.VectorSubcoreMesh(core_axis_name="c", subcore_axis_name="s")


@jax.jit
def _sc_pool(table, idx_flat):
    @functools.partial(
        pl.kernel,
        mesh=_MESH,
        out_type=jax.ShapeDtypeStruct((BATCH, EMBED), jnp.float32),
        scratch_types=[
            pltpu.VMEM((BPW * SEQ,), jnp.int32),
            pltpu.VMEM((NBUF, CB * SEQ, EMBED), jnp.float32),
            pltpu.VMEM((BPW, EMBED), jnp.float32),
        ] + [pltpu.SemaphoreType.DMA] * NBUF,
    )
    def k(table_hbm, idx_hbm, out_hbm, idx_v, rows_v, pooled_v, *sems):
        wid = lax.axis_index("s") * NUM_CORES + lax.axis_index("c")
        pltpu.sync_copy(idx_hbm.at[pl.ds(wid * BPW * SEQ, BPW * SEQ)], idx_v)
        halves = ((0, 104), (104, 96))

        def gather(step, p):
            for off, n in halves:
                pltpu.make_async_copy(
                    table_hbm.at[idx_v.at[pl.ds(step * CB * SEQ + off, n)]],
                    rows_v.at[p, pl.ds(off, n)], sems[p]).start()

        def gather_wait(p):
            for off, n in halves:
                pltpu.make_async_copy(
                    table_hbm.at[idx_v.at[pl.ds(off, n)]],
                    rows_v.at[p, pl.ds(off, n)], sems[p]).wait()

        for p in range(NBUF):
            gather(p, p)

        @pl.loop(0, NSTEPS // NBUF)
        def _step(g):
            for p in range(NBUF):
                step = g * NBUF + p
                gather_wait(p)

                @pl.loop(0, CB)
                def _elem(b):
                    def body(t, acc):
                        for u in range(4):
                            l = 1 + t * 4 + u
                            acc = tuple(
                                acc[j] + rows_v[p, b * SEQ + l,
                                                pl.ds(j * LANES, LANES)]
                                for j in range(NJ))
                        return acc
                    acc0 = tuple(
                        rows_v[p, b * SEQ, pl.ds(j * LANES, LANES)]
                        + rows_v[p, b * SEQ + SEQ - 1, pl.ds(j * LANES, LANES)]
                        for j in range(NJ))
                    acc = lax.fori_loop(0, (SEQ - 2) // 4, body, acc0)
                    for j in range(NJ):
                        pooled_v[step * CB + b, pl.ds(j * LANES, LANES)] = acc[j]

                @pl.when(step + NBUF < NSTEPS)
                def _():
                    gather(step + NBUF, p)

        pltpu.sync_copy(pooled_v, out_hbm.at[pl.ds(wid * BPW, BPW)])

    return k(table, idx_flat)


def _tc_proj_body(p_ref, w_ref, b_ref, o_ref):
    acc = jax.lax.dot_general(
        p_ref[...], w_ref[...],
        dimension_numbers=(((1,), (0,)), ((), ())),
        preferred_element_type=jnp.float32)
    o_ref[...] = acc * (1.0 / SEQ) + b_ref[...]


@jax.jit
def _tc_proj(pooled, W, b2d):
    blk = 2048
    return pl.pallas_call(
        _tc_proj_body,
        grid=(BATCH // blk,),
        in_specs=[
            pl.BlockSpec((blk, EMBED), lambda i: (i, 0)),
            pl.BlockSpec((EMBED, OUT), lambda i: (0, 0)),
            pl.BlockSpec((1, OUT), lambda i: (0, 0)),
        ],
        out_specs=pl.BlockSpec((blk, OUT), lambda i: (i, 0)),
        out_shape=jax.ShapeDtypeStruct((BATCH, OUT), jnp.float32),
    )(pooled, W, b2d)


def kernel(x, table, W, b):
    idx_flat = x.reshape(-1).astype(jnp.int32)
    pooled = _sc_pool(table, idx_flat)
    return _tc_proj(pooled, W, b.reshape(1, OUT))


# speedup vs baseline: 1.0612x; 1.0013x over previous
import functools

import jax
import jax.numpy as jnp
from jax import lax
from jax.experimental import pallas as pl
from jax.experimental.pallas import tpu as pltpu
from jax.experimental.pallas import tpu_sc as plsc

BATCH = 4096
SEQ = 50
EMBED = 128
OUT = 512

NUM_CORES = 2
NUM_SUBCORES = 16
NUM_WORKERS = NUM_CORES * NUM_SUBCORES
BPW = BATCH // NUM_WORKERS
CB = 4
NBUF = 4
NSTEPS = BPW // CB
LANES = 16
NJ = EMBED // LANES

_MESH = plsc.VectorSubcoreMesh(core_axis_name="c", subcore_axis_name="s")


@jax.jit
def _sc_pool(table, idx_flat):
    @functools.partial(
        pl.kernel,
        mesh=_MESH,
        out_type=jax.ShapeDtypeStruct((BATCH, EMBED), jnp.float32),
        scratch_types=[
            pltpu.VMEM((BPW * SEQ,), jnp.int32),
            pltpu.VMEM((NBUF, CB * SEQ, EMBED), jnp.float32),
            pltpu.VMEM((BPW, EMBED), jnp.float32),
        ] + [pltpu.SemaphoreType.DMA] * NBUF,
    )
    def k(table_hbm, idx_hbm, out_hbm, idx_v, rows_v, pooled_v, *sems):
        wid = lax.axis_index("s") * NUM_CORES + lax.axis_index("c")
        pltpu.sync_copy(idx_hbm.at[pl.ds(wid * BPW * SEQ, BPW * SEQ)], idx_v)
        halves = ((0, 104), (104, 96))

        def gather(step, p):
            for off, n in halves:
                pltpu.make_async_copy(
                    table_hbm.at[idx_v.at[pl.ds(step * CB * SEQ + off, n)]],
                    rows_v.at[p, pl.ds(off, n)], sems[p]).start()

        def gather_wait(p):
            for off, n in halves:
                pltpu.make_async_copy(
                    table_hbm.at[idx_v.at[pl.ds(off, n)]],
                    rows_v.at[p, pl.ds(off, n)], sems[p]).wait()

        for p in range(NBUF):
            gather(p, p)

        @pl.loop(0, NSTEPS // NBUF)
        def _step(g):
            for p in range(NBUF):
                step = g * NBUF + p
                gather_wait(p)

                @pl.loop(0, CB)
                def _elem(b):
                    def body(t, acc):
                        for u in range(4):
                            l = 1 + t * 4 + u
                            acc = tuple(
                                acc[j] + rows_v[p, b * SEQ + l,
                                                pl.ds(j * LANES, LANES)]
                                for j in range(NJ))
                        return acc
                    acc0 = tuple(
                        rows_v[p, b * SEQ, pl.ds(j * LANES, LANES)]
                        + rows_v[p, b * SEQ + SEQ - 1, pl.ds(j * LANES, LANES)]
                        for j in range(NJ))
                    acc = lax.fori_loop(0, (SEQ - 2) // 4, body, acc0)
                    for j in range(NJ):
                        pooled_v[step * CB + b, pl.ds(j * LANES, LANES)] = acc[j]

                @pl.when(step + NBUF < NSTEPS)
                def _():
                    gather(step + NBUF, p)

        pltpu.sync_copy(pooled_v, out_hbm.at[pl.ds(wid * BPW, BPW)])

    return k(table, idx_flat)


def _tc_proj_body(p_ref, w_ref, b_ref, o_ref):
    acc = jax.lax.dot_general(
        p_ref[...], w_ref[...],
        dimension_numbers=(((1,), (0,)), ((), ())),
        preferred_element_type=jnp.float32)
    o_ref[...] = acc * (1.0 / SEQ) + b_ref[...]


@jax.jit
def _tc_proj(pooled, W, b2d):
    blk = 2048
    return pl.pallas_call(
        _tc_proj_body,
        grid=(BATCH // blk,),
        in_specs=[
            pl.BlockSpec((blk, EMBED), lambda i: (i, 0)),
            pl.BlockSpec((EMBED, OUT), lambda i: (0, 0)),
            pl.BlockSpec((1, OUT), lambda i: (0, 0)),
        ],
        out_specs=pl.BlockSpec((blk, OUT), lambda i: (i, 0)),
        out_shape=jax.ShapeDtypeStruct((BATCH, OUT), jnp.float32),
    )(pooled, W, b2d)


def kernel(x, table, W, b):
    idx_flat = x.reshape(-1).astype(jnp.int32)
    pooled = _sc_pool(table, idx_flat)
    return _tc_proj(pooled, W, b.reshape(1, OUT))
